# baseline (speedup 1.0000x reference)
import functools

import jax
import jax.numpy as jnp
from jax import lax
from jax.experimental import pallas as pl
from jax.experimental.pallas import tpu as pltpu
from jax.experimental.pallas import tpu_sc as plsc

_D = 64
_B = 4096 * 200

_info = plsc.get_sparse_core_info()
_NC, _NS = _info.num_cores, _info.num_subcores
_NW = _NC * _NS
_BPW = _B // _NW
_CHUNK = 256
_NCHUNK = _BPW // _CHUNK


def _sc_gather(idx_hbm, table_hbm, out_hbm, idx_v, idx2_v, wide_v, sel_v,
               sem_idx, sem_g, sem_out):
    wid = lax.axis_index("s") * _NC + lax.axis_index("c")
    base = wid * _BPW

    def chunk(ci, _):
        off = pl.multiple_of(base + ci * _CHUNK, _CHUNK)
        pltpu.async_copy(idx_hbm.at[pl.ds(off, _CHUNK)], idx_v, sem_idx).wait()
        for j in range(_CHUNK // 16):
            t = idx_v[pl.ds(j * 16, 16)]
            idx2_v[pl.ds(j * 16, 16)] = lax.shift_right_logical(t, 1)
        cps = []
        for j in range(_CHUNK // 128):
            cps.append(pltpu.async_copy(
                table_hbm.at[idx2_v.at[pl.ds(j * 128, 128)]],
                wide_v.at[pl.ds(j * 128, 128)], sem_g))
        for cp in cps:
            cp.wait()

        def cp_group(g, _):
            bvec = (idx_v[pl.ds(g * 16, 16)] & 1) * 64
            for l in range(16):
                i = g * 16 + l
                b = bvec[l]
                for j in range(4):
                    sel_v[i, pl.ds(j * 16, 16)] = (
                        wide_v[i, pl.ds(b + j * 16, 16)])
            return ()
        lax.fori_loop(0, _CHUNK // 16, cp_group, ())

        pltpu.async_copy(sel_v, out_hbm.at[pl.ds(off, _CHUNK)], sem_out).wait()
        return ()

    lax.fori_loop(0, _NCHUNK, chunk, ())


@jax.jit
def _embed(token_ids_flat, table128):
    mesh = plsc.VectorSubcoreMesh(core_axis_name="c", subcore_axis_name="s")
    k = functools.partial(
        pl.kernel,
        mesh=mesh,
        out_type=jax.ShapeDtypeStruct((_B, _D), jnp.float32),
        scratch_types=[
            pltpu.VMEM((_CHUNK,), jnp.int32),
            pltpu.VMEM((_CHUNK,), jnp.int32),
            pltpu.VMEM((_CHUNK, 128), jnp.float32),
            pltpu.VMEM((_CHUNK, _D), jnp.float32),
            pltpu.SemaphoreType.DMA,
            pltpu.SemaphoreType.DMA,
            pltpu.SemaphoreType.DMA,
        ],
    )(_sc_gather)
    return k(token_ids_flat, table128)


def kernel(token_ids, weight):
    flat = token_ids.reshape(-1).astype(jnp.int32)
    w128 = weight.reshape(500000, 128)
    out = _embed(flat, w128)
    return out.reshape(token_ids.shape + (weight.shape[1],))
